# baseline (speedup 1.0000x reference)
import functools

import jax
import jax.numpy as jnp
from jax import lax
from jax.experimental import pallas as pl
from jax.experimental.pallas import tpu as pltpu
from jax.experimental.pallas import tpu_sc as plsc

D = 512
T = 4096
S = 50
SP = 56
ST = 48
NT = S - ST
NC = 2
NS = 16
NW = NC * NS
TPW = T // NW
G = 2
NBANK = TPW // G
NPAIR = NBANK // 2
TR = TPW * NT
TC_ = 32
NTCH = TR // TC_

_mesh = plsc.VectorSubcoreMesh(core_axis_name="c", subcore_axis_name="s")


@functools.partial(
    pl.kernel,
    mesh=_mesh,
    out_type=(
        jax.ShapeDtypeStruct((T, S, D), jnp.float32),
        jax.ShapeDtypeStruct((T * NT, D), jnp.float32),
    ),
    scratch_types=[
        pltpu.VMEM((TPW * SP,), jnp.int32),
        pltpu.VMEM((TR,), jnp.int32),
        pltpu.VMEM((2, G, S, D), jnp.float32),
        pltpu.SemaphoreType.DMA,
        pltpu.SemaphoreType.DMA,
    ],
)
def _embed_gather(xp_hbm, xt_hbm, table_hbm, out_hbm, tails_hbm,
                  idx_v, xt_v, rows_v, sem0, sem1):
    wid = lax.axis_index("s") * NC + lax.axis_index("c")
    tok0 = wid * TPW
    pltpu.sync_copy(xp_hbm.at[pl.ds(tok0 * SP, TPW * SP)], idx_v)
    pltpu.sync_copy(xt_hbm.at[pl.ds(wid * TR, TR)], xt_v)
    sems = (sem0, sem1)

    def gather_bank(k, b):
        for g in range(G):
            pltpu.async_copy(
                table_hbm.at[idx_v.at[pl.ds((G * k + g) * SP, S)]],
                rows_v.at[b, g],
                sems[b],
            )

    def wait_bank(k, b):
        for g in range(G):
            pltpu.make_async_copy(
                table_hbm.at[idx_v.at[pl.ds((G * k + g) * SP, S)]],
                rows_v.at[b, g],
                sems[b],
            ).wait()

    gather_bank(0, 0)
    gather_bank(1, 1)

    def step(i, carry):
        for b in range(2):
            k = 2 * i + b
            wait_bank(k, b)
            pltpu.sync_copy(rows_v.at[b], out_hbm.at[pl.ds(tok0 + G * k, G)])

            @pl.when(i < NPAIR - 1)
            def _():
                gather_bank(k + 2, b)

        return carry

    lax.fori_loop(0, NPAIR, step, 0)

    def tbuf(b):
        return rows_v.at[b, 0].at[pl.ds(0, TC_)]

    def tgather(q, b):
        pltpu.async_copy(
            table_hbm.at[xt_v.at[pl.ds(TC_ * q, TC_)]], tbuf(b), sems[b]
        )

    def twait(q, b):
        pltpu.make_async_copy(
            table_hbm.at[xt_v.at[pl.ds(TC_ * q, TC_)]], tbuf(b), sems[b]
        ).wait()

    tgather(0, 0)
    tgather(1, 1)
    for q in range(NTCH):
        b = q % 2
        twait(q, b)
        pltpu.sync_copy(tbuf(b), tails_hbm.at[pl.ds(wid * TR + TC_ * q, TC_)])
        if q + 2 < NTCH:
            tgather(q + 2, b)


def kernel(x, table):
    xp = jnp.pad(x, ((0, 0), (0, SP - S))).reshape(-1)
    xt = x[:, ST:].reshape(-1)
    out, tails = _embed_gather(xp, xt, table)
    return out.at[:, ST:, :].set(tails.reshape(T, NT, D))
